# baseline (speedup 1.0000x reference)
import functools

import jax
import jax.numpy as jnp
from jax import lax
from jax.experimental import pallas as pl
from jax.experimental.pallas import tpu as pltpu
from jax.experimental.pallas import tpu_sc as plsc

B = 16384
C = 1000
NC = 2
NS = 16
NW = NC * NS
RPW = B // NW
GCH = 64
BM = 512
ALPHA = 0.1


def _sc_body(pred_hbm, tgt_hbm, lw_hbm, g_hbm, tgt_v, lw_v, gidx_v, rows_v, sem):
    wid = lax.axis_index("s") * NC + lax.axis_index("c")
    lo = wid * RPW

    with jax.named_scope("stage_tgt"):
        pltpu.sync_copy(tgt_hbm, tgt_v)

    neg1 = jnp.full((16,), -1, jnp.int32)
    for k in range(RPW // 16):
        lw_v[pl.ds(k * 16, 16)] = neg1

    lanes = lax.iota(jnp.int32, 16)

    def scan_step(k, carry):
        t = tgt_v[pl.ds(k * 16, 16)]
        rel = t - lo
        m = (rel >= 0) & (rel < RPW)
        rel_c = jnp.where(m, rel, 0)
        ivals = lanes + k * 16
        plsc.store_scatter(lw_v, [rel_c], ivals, mask=m)
        return carry

    with jax.named_scope("lw_scan"):
        lax.fori_loop(0, B // 16, scan_step, 0)

    pltpu.sync_copy(lw_v, lw_hbm.at[pl.ds(lo, RPW)])

    zero = jnp.zeros((16,), jnp.int32)
    for k in range(RPW // 16):
        gidx_v[pl.ds(k * 16, 16)] = jnp.maximum(lw_v[pl.ds(k * 16, 16)], zero)

    with jax.named_scope("gather"):
        for ck in range(RPW // GCH):
            idx_slice = gidx_v.at[pl.ds(ck * GCH, GCH)]
            pltpu.async_copy(pred_hbm.at[idx_slice], rows_v, sem).wait()
            pltpu.sync_copy(rows_v, g_hbm.at[pl.ds(lo + ck * GCH, GCH)])


def _sc_lw_and_gather(pred, tgt):
    mesh = plsc.VectorSubcoreMesh(core_axis_name="c", subcore_axis_name="s")
    k = pl.kernel(
        _sc_body,
        mesh=mesh,
        compiler_params=pltpu.CompilerParams(needs_layout_passes=False, use_tc_tiling_on_sc=False),
        out_type=[
            jax.ShapeDtypeStruct((B,), jnp.int32),
            jax.ShapeDtypeStruct((B, C), jnp.float32),
        ],
        scratch_types=[
            pltpu.VMEM((B,), jnp.int32),
            pltpu.VMEM((RPW,), jnp.int32),
            pltpu.VMEM((RPW,), jnp.int32),
            pltpu.VMEM((GCH, C), jnp.float32),
            pltpu.SemaphoreType.DMA,
        ],
    )
    return k(pred, tgt)


def _tc_body(pred_ref, g_ref, lw_ref, out_ref):
    i = pl.program_id(0)
    x = pred_ref[...]
    m = jnp.max(x, axis=1, keepdims=True)
    e = jnp.exp(x - m)
    s = jnp.sum(e, axis=1, keepdims=True)
    p = e / s
    p2 = p * p
    a_row = jnp.sum(-ALPHA * p2 * jnp.log(1.0 - p), axis=1)

    g = g_ref[...]
    mg = jnp.max(g, axis=1, keepdims=True)
    sg = jnp.sum(jnp.exp(g - mg), axis=1, keepdims=True)
    lseg = mg + jnp.log(sg)
    w_row = -(1.0 - ALPHA) * jnp.sum(p2 * (g - lseg), axis=1)

    lw = lw_ref[0, 0, :]
    part = jnp.sum(jnp.where(lw >= 0, w_row, a_row))

    @pl.when(i == 0)
    def _():
        out_ref[0, 0] = 0.0

    out_ref[0, 0] += part


def _tc_loss(pred, g, lw3):
    return pl.pallas_call(
        _tc_body,
        grid=(B // BM,),
        in_specs=[
            pl.BlockSpec((BM, C), lambda i: (i, 0)),
            pl.BlockSpec((BM, C), lambda i: (i, 0)),
            pl.BlockSpec((1, 1, BM), lambda i: (i, 0, 0)),
        ],
        out_specs=pl.BlockSpec(memory_space=pltpu.SMEM),
        out_shape=jax.ShapeDtypeStruct((1, 1), jnp.float32),
    )(pred, g, lw3)


@jax.jit
def kernel(pred, target):
    tgt = target.astype(jnp.int32)
    lw, g = _sc_lw_and_gather(pred, tgt)
    lw3 = lw.reshape(B // BM, 1, BM)
    out = _tc_loss(pred, g, lw3)
    return out[0, 0]


# speedup vs baseline: 2.7741x; 2.7741x over previous
import jax
import jax.numpy as jnp
from jax import lax
from jax.experimental import pallas as pl
from jax.experimental.pallas import tpu as pltpu
from jax.experimental.pallas import tpu_sc as plsc

B = 16384
C = 1000
CP = 1024
NC = 2
NS = 16
NW = NC * NS
RPW = B // NW
GCH = 32
NCK = RPW // GCH
BM = 512
ALPHA = 0.1


def _sc_body(pred_hbm, tgt_hbm, lw_hbm, g_hbm, tgt_v, lw_v, gidx_v,
             rows_a, rows_b, sem_a, sem_b):
    wid = lax.axis_index("s") * NC + lax.axis_index("c")
    lo = wid * RPW

    with jax.named_scope("stage_tgt"):
        pltpu.sync_copy(tgt_hbm, tgt_v)

    neg1 = jnp.full((16,), -1, jnp.int32)
    for k in range(RPW // 16):
        lw_v[pl.ds(k * 16, 16)] = neg1

    lanes = lax.iota(jnp.int32, 16)

    def scan_step(k, carry):
        t = tgt_v[pl.ds(k * 16, 16)]
        rel = t - lo
        m = (rel >= 0) & (rel < RPW)
        rel_c = jnp.where(m, rel, 0)
        ivals = lanes + k * 16
        plsc.store_scatter(lw_v, [rel_c], ivals, mask=m)
        return carry

    with jax.named_scope("lw_scan"):
        lax.fori_loop(0, B // 16, scan_step, 0)

    pltpu.sync_copy(lw_v, lw_hbm.at[pl.ds(lo, RPW)])

    for k in range(RPW // 16):
        v = lw_v[pl.ds(k * 16, 16)]
        own = lanes + (lo + k * 16)
        gidx_v[pl.ds(k * 16, 16)] = jnp.where(v >= 0, v, own)

    bufs = (rows_a, rows_b)
    sems = (sem_a, sem_b)
    with jax.named_scope("gather"):
        handles = [None, None]
        for ck in range(NCK):
            b = ck & 1
            idx_slice = gidx_v.at[pl.ds(ck * GCH, GCH)]
            handles[b] = pltpu.async_copy(pred_hbm.at[idx_slice], bufs[b], sems[b])
            if ck > 0:
                pb = 1 - b
                handles[pb].wait()
                pltpu.sync_copy(bufs[pb], g_hbm.at[pl.ds(lo + (ck - 1) * GCH, GCH)])
        lb = (NCK - 1) & 1
        handles[lb].wait()
        pltpu.sync_copy(bufs[lb], g_hbm.at[pl.ds(lo + (NCK - 1) * GCH, GCH)])


def _sc_lw_and_gather(pred_p, tgt):
    mesh = plsc.VectorSubcoreMesh(core_axis_name="c", subcore_axis_name="s")
    k = pl.kernel(
        _sc_body,
        mesh=mesh,
        compiler_params=pltpu.CompilerParams(needs_layout_passes=False),
        out_type=[
            jax.ShapeDtypeStruct((B,), jnp.int32),
            jax.ShapeDtypeStruct((B, CP), jnp.float32),
        ],
        scratch_types=[
            pltpu.VMEM((B,), jnp.int32),
            pltpu.VMEM((RPW,), jnp.int32),
            pltpu.VMEM((RPW,), jnp.int32),
            pltpu.VMEM((GCH, CP), jnp.float32),
            pltpu.VMEM((GCH, CP), jnp.float32),
            pltpu.SemaphoreType.DMA,
            pltpu.SemaphoreType.DMA,
        ],
    )
    return k(pred_p, tgt)


def _tc_body(pred_ref, g_ref, lw_ref, out_ref):
    i = pl.program_id(0)
    colmask = lax.broadcasted_iota(jnp.int32, (1, CP), 1) < C

    x = pred_ref[...]
    m = jnp.max(x, axis=1, keepdims=True)
    e = jnp.where(colmask, jnp.exp(x - m), 0.0)
    s = jnp.sum(e, axis=1, keepdims=True)
    p = e / s
    p2 = p * p
    a_row = jnp.sum(-ALPHA * p2 * jnp.log(1.0 - p), axis=1)

    g = g_ref[...]
    mg = jnp.max(g, axis=1, keepdims=True)
    eg = jnp.where(colmask, jnp.exp(g - mg), 0.0)
    sg = jnp.sum(eg, axis=1, keepdims=True)
    lseg = mg + jnp.log(sg)
    w_row = -(1.0 - ALPHA) * jnp.sum(p2 * (g - lseg), axis=1)

    lw = lw_ref[0, 0, :]
    part = jnp.sum(jnp.where(lw >= 0, w_row, a_row))

    @pl.when(i == 0)
    def _():
        out_ref[0, 0] = 0.0

    out_ref[0, 0] += part


def _tc_loss(pred_p, g, lw3):
    return pl.pallas_call(
        _tc_body,
        grid=(B // BM,),
        in_specs=[
            pl.BlockSpec((BM, CP), lambda i: (i, 0)),
            pl.BlockSpec((BM, CP), lambda i: (i, 0)),
            pl.BlockSpec((1, 1, BM), lambda i: (i, 0, 0)),
        ],
        out_specs=pl.BlockSpec(memory_space=pltpu.SMEM),
        out_shape=jax.ShapeDtypeStruct((1, 1), jnp.float32),
    )(pred_p, g, lw3)


@jax.jit
def kernel(pred, target):
    tgt = target.astype(jnp.int32)
    pred_p = jnp.pad(pred, ((0, 0), (0, CP - C)))
    lw, g = _sc_lw_and_gather(pred_p, tgt)
    lw3 = lw.reshape(B // BM, 1, BM)
    out = _tc_loss(pred_p, g, lw3)
    return out[0, 0]
